# baseline (speedup 1.0000x reference)
import functools

import jax
import jax.numpy as jnp
from jax import lax
from jax.experimental import pallas as pl
from jax.experimental.pallas import tpu as pltpu
from jax.experimental.pallas import tpu_sc as plsc

N = 10000
E = 320000
D = 128


_ROWS = 1000


def _bf16_bits(x):
    xi = lax.bitcast_convert_type(x, jnp.int32)
    return ((xi + 0x7FFF + ((xi >> 16) & 1)) >> 16) & 0xFFFF


def _pack_pair(f):
    xi = lax.bitcast_convert_type(f[:, D // 2:], jnp.int32)
    hi = (xi + 0x7FFF + ((xi >> 16) & 1)) & jnp.int32(-65536)
    return _bf16_bits(f[:, : D // 2]) | hi


def _tc_tables_body(h_ref, w1a_ref, w1b_ref, b1_ref, w2a_ref, t_ref):
    hblk = h_ref[...]
    dn = (((1,), (1,)), ((), ()))
    w2a = w2a_ref[...]
    af = (
        lax.dot_general(hblk, w1a_ref[...], dn,
                        preferred_element_type=jnp.float32,
                        precision=lax.Precision.HIGHEST)
        + b1_ref[...]
    ) * w2a
    bf = lax.dot_general(hblk, w1b_ref[...], dn,
                         preferred_element_type=jnp.float32,
                         precision=lax.Precision.HIGHEST) * w2a
    t_ref[...] = jnp.concatenate([_pack_pair(af), _pack_pair(bf)], axis=1)


def _build_tables(h, w1a, w1b, b1, w2a):
    grid = N // _ROWS
    return pl.pallas_call(
        _tc_tables_body,
        grid=(grid,),
        in_specs=[
            pl.BlockSpec((_ROWS, D), lambda i: (i, 0)),
            pl.BlockSpec((D, D), lambda i: (0, 0)),
            pl.BlockSpec((D, D), lambda i: (0, 0)),
            pl.BlockSpec((1, D), lambda i: (0, 0)),
            pl.BlockSpec((1, D), lambda i: (0, 0)),
        ],
        out_specs=pl.BlockSpec((_ROWS, D), lambda i: (i, 0)),
        out_shape=jax.ShapeDtypeStruct((N, D), jnp.int32),
    )(h, w1a, w1b, b1, w2a)


_NC = 2
_NS = 16
_NW = _NC * _NS
_EW = E // _NW
_C = 80
_R = _EW // _C
_K = D // 16


def _sc_edge_body(t_hbm, src_hbm, dst_hbm, wv_hbm, tail_hbm, out_hbm,
                  srcv_all, dstv_all, arows, brows, outv, wv_v, tail_v,
                  mblk, sem0, sem1):
    wid = lax.axis_index("s") * _NC + lax.axis_index("c")
    base = wid * _EW
    pltpu.sync_copy(wv_hbm, wv_v)
    pltpu.sync_copy(tail_hbm, tail_v)
    pltpu.sync_copy(src_hbm.at[pl.ds(base, _EW)], srcv_all)
    pltpu.sync_copy(dst_hbm.at[pl.ds(base, _EW)], dstv_all)
    b2v = tail_v[...]
    sm = [wv_v[pl.ds(16 * k, 16)] for k in range(D // 32)]
    lane16 = lax.iota(jnp.int32, 16) * 16
    sems = (sem0, sem1)

    def fire(r, b):
        pltpu.async_copy(t_hbm.at[srcv_all.at[pl.ds(r * _C, _C)]],
                         arows.at[b], sems[b])
        pltpu.async_copy(t_hbm.at[dstv_all.at[pl.ds(r * _C, _C)]],
                         brows.at[b], sems[b])

    def wait_round(r, b):
        pltpu.make_async_copy(t_hbm.at[srcv_all.at[pl.ds(r * _C, _C)]],
                              arows.at[b], sems[b]).wait()
        pltpu.make_async_copy(t_hbm.at[dstv_all.at[pl.ds(r * _C, _C)]],
                              brows.at[b], sems[b]).wait()

    def compute(r, b):
        def scatter_group(g):
            mo = (g % 2) * 256
            accs = [None] * 16
            for k in range(D // 32):
                for i in range(16):
                    e = g * 16 + i
                    aw = arows[b, e, pl.ds(16 * k, 16)]
                    bw = brows[b, e, pl.ds(D // 2 + 16 * k, 16)]
                    t = jnp.maximum(plsc.bitcast(aw, jnp.bfloat16)
                                    + plsc.bitcast(bw, jnp.bfloat16),
                                    jnp.bfloat16(0.0))
                    w = plsc.bitcast(t, jnp.int32) ^ sm[k]
                    tl = plsc.bitcast(w << 16, jnp.float32)
                    th = plsc.bitcast(w & jnp.int32(-65536), jnp.float32)
                    part = tl + th
                    accs[i] = part if accs[i] is None else accs[i] + part
            for i in range(16):
                plsc.store_scatter(mblk, [mo + lane16 + i], accs[i])

        def reduce_group(g):
            mo = (g % 2) * 256
            rows = [mblk[pl.ds(mo + 16 * i, 16)] for i in range(16)]
            rows.append(b2v)
            while len(rows) > 1:
                rows = [rows[i] + rows[i + 1]
                        for i in range(0, len(rows) - 1, 2)] \
                    + ([rows[-1]] if len(rows) % 2 else [])
            outv[pl.ds(r * _C + g * 16, 16)] = rows[0]

        ngroups = _C // 16
        scatter_group(0)
        for g in range(1, ngroups):
            scatter_group(g)
            reduce_group(g - 1)
        reduce_group(ngroups - 1)

    fire(0, 0)

    def pair_body(p, c):
        r0 = 2 * p
        fire(r0 + 1, 1)
        wait_round(r0, 0)
        compute(r0, 0)
        fire(r0 + 2, 0)
        wait_round(r0 + 1, 1)
        compute(r0 + 1, 1)
        return c

    lax.fori_loop(0, (_R - 1) // 2, pair_body, 0)
    wait_round(_R - 1, 0)
    compute(_R - 1, 0)
    pltpu.sync_copy(outv.at[pl.ds(0, _EW)], out_hbm.at[pl.ds(base, _EW)])


def _score_edges(t_tab, src, dst, wv, tail):
    mesh = plsc.VectorSubcoreMesh(core_axis_name="c", subcore_axis_name="s")
    f = pl.kernel(
        _sc_edge_body,
        out_type=jax.ShapeDtypeStruct((E,), jnp.float32),
        mesh=mesh,
        compiler_params=pltpu.CompilerParams(needs_layout_passes=False),
        scratch_types=[
            pltpu.VMEM((_EW,), jnp.int32),
            pltpu.VMEM((_EW,), jnp.int32),
            pltpu.VMEM((2, _C, D), jnp.int32),
            pltpu.VMEM((2, _C, D), jnp.int32),
            pltpu.VMEM((_EW,), jnp.float32),
            pltpu.VMEM((D // 2,), jnp.int32),
            pltpu.VMEM((16,), jnp.float32),
            pltpu.VMEM((512,), jnp.float32),
            pltpu.SemaphoreType.DMA,
            pltpu.SemaphoreType.DMA,
        ],
    )
    return f(t_tab, src, dst, wv, tail)


def kernel(h, edge_index, W1, b1, W2, b2):
    w1a = W1[:, :D]
    w1b = W1[:, D:]
    b1r = b1.reshape(1, D)
    w2a = jnp.abs(W2).reshape(1, D)
    t_tab = _build_tables(h, w1a, w1b, b1r, w2a)
    src = edge_index[0]
    dst = edge_index[1]
    neg = W2[0] < 0
    wv = (jnp.where(neg[: D // 2], jnp.int32(0x8000), 0)
          | jnp.where(neg[D // 2:], jnp.int32(-2147483648), 0))
    tail = jnp.full((16,), b2[0], jnp.float32)
    return _score_edges(t_tab, src, dst, wv, tail)


# speedup vs baseline: 1.2875x; 1.2875x over previous
import jax
import jax.numpy as jnp
from jax import lax
from jax.experimental import pallas as pl
from jax.experimental.pallas import tpu as pltpu
from jax.experimental.pallas import tpu_sc as plsc

N = 10000
E = 320000
D = 128


_ROWS = 1000


def _bf16_bits(x):
    xi = lax.bitcast_convert_type(x, jnp.int32)
    return ((xi + 0x7FFF + ((xi >> 16) & 1)) >> 16) & 0xFFFF


def _pack_pair(f):
    xi = lax.bitcast_convert_type(f[:, D // 2:], jnp.int32)
    hi = (xi + 0x7FFF + ((xi >> 16) & 1)) & jnp.int32(-65536)
    return _bf16_bits(f[:, : D // 2]) | hi


def _tc_tables_body(h_ref, w1a_ref, w1b_ref, b1_ref, w2a_ref, t_ref):
    hblk = h_ref[...]
    dn = (((1,), (1,)), ((), ()))
    w2a = w2a_ref[...]
    af = (
        lax.dot_general(hblk, w1a_ref[...], dn,
                        preferred_element_type=jnp.float32,
                        precision=lax.Precision.HIGHEST)
        + b1_ref[...]
    ) * w2a
    bf = lax.dot_general(hblk, w1b_ref[...], dn,
                         preferred_element_type=jnp.float32,
                         precision=lax.Precision.HIGHEST) * w2a
    t_ref[...] = jnp.concatenate([_pack_pair(af), _pack_pair(bf)], axis=1)


def _build_tables(h, w1a, w1b, b1, w2a):
    grid = N // _ROWS
    return pl.pallas_call(
        _tc_tables_body,
        grid=(grid,),
        in_specs=[
            pl.BlockSpec((_ROWS, D), lambda i: (i, 0)),
            pl.BlockSpec((D, D), lambda i: (0, 0)),
            pl.BlockSpec((D, D), lambda i: (0, 0)),
            pl.BlockSpec((1, D), lambda i: (0, 0)),
            pl.BlockSpec((1, D), lambda i: (0, 0)),
        ],
        out_specs=pl.BlockSpec((_ROWS, D), lambda i: (i, 0)),
        out_shape=jax.ShapeDtypeStruct((N, D), jnp.int32),
    )(h, w1a, w1b, b1, w2a)


_NC = 2
_NS = 16
_NW = _NC * _NS
_EW = E // _NW
_C = 80
_R = _EW // _C


def _sc_edge_body(t_hbm, src_hbm, dst_hbm, wv_hbm, tail_hbm, out_hbm,
                  srcv_all, dstv_all, arows, brows, outv, wv_v, tail_v,
                  mblk, sem0, sem1):
    wid = lax.axis_index("s") * _NC + lax.axis_index("c")
    base = wid * _EW
    pltpu.sync_copy(wv_hbm, wv_v)
    pltpu.sync_copy(tail_hbm, tail_v)
    pltpu.sync_copy(src_hbm.at[pl.ds(base, _EW)], srcv_all)
    pltpu.sync_copy(dst_hbm.at[pl.ds(base, _EW)], dstv_all)
    b2v = tail_v[...]
    sm = [wv_v[pl.ds(16 * k, 16)] for k in range(D // 32)]
    lane16 = lax.iota(jnp.int32, 16) * 16
    sems = (sem0, sem1)

    def fire(r, b):
        pltpu.async_copy(t_hbm.at[srcv_all.at[pl.ds(r * _C, _C)]],
                         arows.at[b], sems[b])
        pltpu.async_copy(t_hbm.at[dstv_all.at[pl.ds(r * _C, _C)]],
                         brows.at[b], sems[b])

    def wait_round(r, b):
        pltpu.make_async_copy(t_hbm.at[srcv_all.at[pl.ds(r * _C, _C)]],
                              arows.at[b], sems[b]).wait()
        pltpu.make_async_copy(t_hbm.at[dstv_all.at[pl.ds(r * _C, _C)]],
                              brows.at[b], sems[b]).wait()

    def compute(r, b):
        def group_body(g, c):
            mo = 0
            accs = [None] * 16
            for k in range(D // 32):
                for i in range(16):
                    e = g * 16 + i
                    aw = arows[b, e, pl.ds(16 * k, 16)]
                    bw = brows[b, e, pl.ds(D // 2 + 16 * k, 16)]
                    t = jnp.maximum(plsc.bitcast(aw, jnp.bfloat16)
                                    + plsc.bitcast(bw, jnp.bfloat16),
                                    jnp.bfloat16(0.0))
                    w = plsc.bitcast(t, jnp.int32) ^ sm[k]
                    tl = plsc.bitcast(w << 16, jnp.float32)
                    th = plsc.bitcast(w & jnp.int32(-65536), jnp.float32)
                    part = tl + th
                    accs[i] = part if accs[i] is None else accs[i] + part
            for i in range(16):
                plsc.store_scatter(mblk, [mo + lane16 + i], accs[i])
            rows = [mblk[pl.ds(mo + 16 * i, 16)] for i in range(16)]
            rows.append(b2v)
            while len(rows) > 1:
                rows = [rows[i] + rows[i + 1]
                        for i in range(0, len(rows) - 1, 2)] \
                    + ([rows[-1]] if len(rows) % 2 else [])
            outv[pl.ds(r * _C + g * 16, 16)] = rows[0]
            return c

        lax.fori_loop(0, _C // 16, group_body, 0)

    fire(0, 0)

    def pair_body(p, c):
        r0 = 2 * p
        fire(r0 + 1, 1)
        wait_round(r0, 0)
        compute(r0, 0)
        fire(r0 + 2, 0)
        wait_round(r0 + 1, 1)
        compute(r0 + 1, 1)
        return c

    lax.fori_loop(0, (_R - 1) // 2, pair_body, 0)
    wait_round(_R - 1, 0)
    compute(_R - 1, 0)
    pltpu.sync_copy(outv.at[pl.ds(0, _EW)], out_hbm.at[pl.ds(base, _EW)])


def _score_edges(t_tab, src, dst, wv, tail):
    mesh = plsc.VectorSubcoreMesh(core_axis_name="c", subcore_axis_name="s")
    f = pl.kernel(
        _sc_edge_body,
        out_type=jax.ShapeDtypeStruct((E,), jnp.float32),
        mesh=mesh,
        compiler_params=pltpu.CompilerParams(needs_layout_passes=False),
        scratch_types=[
            pltpu.VMEM((_EW,), jnp.int32),
            pltpu.VMEM((_EW,), jnp.int32),
            pltpu.VMEM((2, _C, D), jnp.int32),
            pltpu.VMEM((2, _C, D), jnp.int32),
            pltpu.VMEM((_EW,), jnp.float32),
            pltpu.VMEM((D // 2,), jnp.int32),
            pltpu.VMEM((16,), jnp.float32),
            pltpu.VMEM((256,), jnp.float32),
            pltpu.SemaphoreType.DMA,
            pltpu.SemaphoreType.DMA,
        ],
    )
    return f(t_tab, src, dst, wv, tail)


def kernel(h, edge_index, W1, b1, W2, b2):
    w1a = W1[:, :D]
    w1b = W1[:, D:]
    b1r = b1.reshape(1, D)
    w2a = jnp.abs(W2).reshape(1, D)
    t_tab = _build_tables(h, w1a, w1b, b1r, w2a)
    src = edge_index[0]
    dst = edge_index[1]
    neg = W2[0] < 0
    wv = (jnp.where(neg[: D // 2], jnp.int32(0x8000), 0)
          | jnp.where(neg[D // 2:], jnp.int32(-2147483648), 0))
    tail = jnp.full((16,), b2[0], jnp.float32)
    return _score_edges(t_tab, src, dst, wv, tail)
